# baseline (speedup 1.0000x reference)
import functools

import jax
import jax.numpy as jnp
from jax import lax
from jax.experimental import pallas as pl
from jax.experimental.pallas import tpu as pltpu
from jax.experimental.pallas import tpu_sc as plsc

N = 1_000_000
N_CLASS = 100
N_BINS = 15
PAD_C = 128
NW = 32
NS = 4
NCH = NW * NS
SCHUNK = 7_824
CHUNK = NS * SCHUNK
N_PAD = NCH * SCHUNK
STEPS = SCHUNK // 16
CELLS = 16 * PAD_C

_mesh = plsc.VectorSubcoreMesh(core_axis_name="c", subcore_axis_name="s")
_sc_params = pltpu.CompilerParams(needs_layout_passes=False)


def _wid():
    return lax.axis_index("c") * 16 + lax.axis_index("s")


def _div15(cnt):
    m0 = (cnt.astype(jnp.float32) * jnp.float32(1.0 / 15.0)).astype(jnp.int32)
    return m0 + (cnt >= (m0 + 1) * 15).astype(jnp.int32) \
              - (cnt < m0 * 15).astype(jnp.int32)


@functools.partial(
    pl.kernel,
    out_type=jax.ShapeDtypeStruct((NCH * PAD_C,), jnp.int32),
    mesh=_mesh,
    compiler_params=_sc_params,
    scratch_types=[pltpu.VMEM((CHUNK,), jnp.int32),
                   pltpu.VMEM((NS * PAD_C,), jnp.int32)],
)
def _hist_kernel(lab_hbm, hist_out, lab_v, h_v):
    w = _wid()
    pltpu.sync_copy(lab_hbm.at[pl.ds(w * CHUNK, CHUNK)], lab_v)
    zi = jnp.zeros((16,), jnp.int32)

    def zero(i, _):
        h_v[pl.ds(i * 16, 16)] = zi
        return 0

    lax.fori_loop(0, NS * PAD_C // 16, zero, 0)
    ones = jnp.ones((16,), jnp.int32)

    def body(i, _):
        o = i * 16
        for s in range(NS):
            lab = lab_v[pl.ds(s * SCHUNK + o, 16)]
            if s:
                lab = lab + jnp.full((16,), s * PAD_C, jnp.int32)
            plsc.addupdate_scatter(h_v, [lab], ones)
        return 0

    lax.fori_loop(0, STEPS, body, 0)
    pltpu.sync_copy(h_v, hist_out.at[pl.ds(w * NS * PAD_C, NS * PAD_C)])


@functools.partial(
    pl.kernel,
    out_type=(jax.ShapeDtypeStruct((NW, CELLS), jnp.float32),
              jax.ShapeDtypeStruct((NW, CELLS), jnp.float32)),
    mesh=_mesh,
    compiler_params=_sc_params,
    scratch_types=[
        pltpu.VMEM((CHUNK,), jnp.int32),
        pltpu.VMEM((CHUNK,), jnp.float32),
        pltpu.VMEM((CHUNK,), jnp.float32),
        pltpu.VMEM((NCH * PAD_C,), jnp.int32),
        pltpu.VMEM((NS * PAD_C,), jnp.int32),
        pltpu.VMEM((PAD_C,), jnp.float32),
        pltpu.VMEM((CELLS,), jnp.float32),
        pltpu.VMEM((CELLS,), jnp.float32),
        pltpu.SemaphoreType.DMA,
        pltpu.SemaphoreType.DMA,
        pltpu.SemaphoreType.DMA,
    ],
)
def _main_kernel(cf_hbm, ht_hbm, lab_hbm, hist_hbm, accp_out, acch_out,
                 lab_v, cf_v, ht_v, hist_v, cnt_v, rcp_v,
                 accp_v, acch_v, sem1, sem2, sem3):
    w = _wid()
    base = w * CHUNK
    cp1 = pltpu.async_copy(lab_hbm.at[pl.ds(base, CHUNK)], lab_v, sem1)
    cp2 = pltpu.async_copy(cf_hbm.at[pl.ds(base, CHUNK)], cf_v, sem2)
    cp3 = pltpu.async_copy(ht_hbm.at[pl.ds(base, CHUNK)], ht_v, sem3)
    pltpu.sync_copy(hist_hbm, hist_v)

    zi = jnp.zeros((16,), jnp.int32)
    c0 = NS * w
    for j in range(PAD_C // 16):
        def acc_v(v, carry):
            tot, off = carry
            hv = hist_v[pl.ds(v * PAD_C + j * 16, 16)]
            sel = (v < c0).astype(jnp.int32)
            return (tot + hv, off + hv * sel)

        tot, off = lax.fori_loop(0, NCH, acc_v, (zi, zi))
        for s in range(NS):
            cnt_v[pl.ds(s * PAD_C + j * 16, 16)] = off
            if s + 1 < NS:
                off = off + hist_v[pl.ds((c0 + s) * PAD_C + j * 16, 16)]
        m = _div15(tot).astype(jnp.float32)
        rcp_v[pl.ds(j * 16, 16)] = (jnp.float32(1.0) / m) \
            * jnp.float32(1.0 + 2.0 ** -22)

    zf = jnp.zeros((16,), jnp.float32)

    def zero(i, _):
        accp_v[pl.ds(i * 16, 16)] = zf
        acch_v[pl.ds(i * 16, 16)] = zf
        return 0

    lax.fori_loop(0, CELLS // 16, zero, 0)
    cp1.wait()
    cp2.wait()
    cp3.wait()

    ones = jnp.ones((16,), jnp.int32)
    tmask = jnp.ones((16,), jnp.bool_)

    def stream(s, lab, cf, ht):
        labc = lab if s == 0 else lab + jnp.full((16,), s * PAD_C, jnp.int32)
        cbase = plsc.load_gather(cnt_v, [labc])
        occ, _ = plsc.scan_count(lab, mask=tmask)
        rank = cbase + occ.astype(jnp.int32) - 1
        plsc.addupdate_scatter(cnt_v, [labc], ones)
        r = plsc.load_gather(rcp_v, [lab])
        q = (rank.astype(jnp.float32) * r).astype(jnp.int32)
        q = jnp.clip(q, 0, 15)
        idx = q * PAD_C + lab
        plsc.addupdate_scatter(accp_v, [idx], cf)
        plsc.addupdate_scatter(acch_v, [idx], ht)

    def body(i, _):
        o = i * 16
        for s in range(NS):
            so = s * SCHUNK + o
            stream(s, lab_v[pl.ds(so, 16)], cf_v[pl.ds(so, 16)],
                   ht_v[pl.ds(so, 16)])
        return 0

    lax.fori_loop(0, STEPS, body, 0)
    pltpu.sync_copy(accp_v, accp_out.at[w])
    pltpu.sync_copy(acch_v, acch_out.at[w])


def _final_body(accp_ref, acch_ref, hist_ref, out_ref):
    p = accp_ref[0]
    h = acch_ref[0]
    cnt = hist_ref[0]
    for v in range(1, NW):
        p = p + accp_ref[v]
        h = h + acch_ref[v]
    for v in range(1, NCH):
        cnt = cnt + hist_ref[v]
    mass = _div15(cnt).astype(jnp.float32)
    coef = jnp.float32(1.0) / (mass * jnp.float32(N_BINS * N_CLASS))
    cls = lax.broadcasted_iota(jnp.int32, (1, PAD_C), 1)
    coef = jnp.where(cls < N_CLASS, coef, jnp.float32(0.0))
    d = jnp.abs(p - h) * coef
    b = lax.broadcasted_iota(jnp.int32, (16, PAD_C), 0)
    d = jnp.where(b < N_BINS, d, jnp.float32(0.0))
    out_ref[...] = jnp.sum(d).reshape(1, 1)


def kernel(confidences, hits, labels):
    pad = N_PAD - N
    cf = jnp.concatenate([confidences, jnp.zeros((pad,), jnp.float32)])
    ht = jnp.concatenate([hits, jnp.zeros((pad,), jnp.float32)])
    lab = jnp.concatenate([labels, jnp.full((pad,), PAD_C - 1, jnp.int32)])
    hist = _hist_kernel(lab)
    accp, acch = _main_kernel(cf, ht, lab, hist)
    out = pl.pallas_call(
        _final_body,
        out_shape=jax.ShapeDtypeStruct((1, 1), jnp.float32),
    )(accp.reshape(NW, 16, PAD_C), acch.reshape(NW, 16, PAD_C),
      hist.reshape(NCH, 1, PAD_C))
    return out[0, 0]


# speedup vs baseline: 1.0442x; 1.0442x over previous
import functools

import jax
import jax.numpy as jnp
from jax import lax
from jax.experimental import pallas as pl
from jax.experimental.pallas import tpu as pltpu
from jax.experimental.pallas import tpu_sc as plsc

N = 1_000_000
N_CLASS = 100
N_BINS = 15
PAD_C = 128
NW = 32
NS = 4
NCH = NW * NS
SCHUNK = 7_824
CHUNK = NS * SCHUNK
N_PAD = NCH * SCHUNK
STEPS = SCHUNK // 16
CELLS = 16 * PAD_C

_mesh = plsc.VectorSubcoreMesh(core_axis_name="c", subcore_axis_name="s")
_sc_params = pltpu.CompilerParams(needs_layout_passes=False)


def _wid():
    return lax.axis_index("c") * 16 + lax.axis_index("s")


def _div15(cnt):
    m0 = (cnt.astype(jnp.float32) * jnp.float32(1.0 / 15.0)).astype(jnp.int32)
    return m0 + (cnt >= (m0 + 1) * 15).astype(jnp.int32) \
              - (cnt < m0 * 15).astype(jnp.int32)


@functools.partial(
    pl.kernel,
    out_type=jax.ShapeDtypeStruct((NCH * PAD_C,), jnp.int32),
    mesh=_mesh,
    compiler_params=_sc_params,
    scratch_types=[pltpu.VMEM((CHUNK,), jnp.int32),
                   pltpu.VMEM((NS * PAD_C,), jnp.int32)],
)
def _hist_kernel(lab_hbm, hist_out, lab_v, h_v):
    w = _wid()
    pltpu.sync_copy(lab_hbm.at[pl.ds(w * CHUNK, CHUNK)], lab_v)
    zi = jnp.zeros((16,), jnp.int32)

    def zero(i, _):
        h_v[pl.ds(i * 16, 16)] = zi
        return 0

    lax.fori_loop(0, NS * PAD_C // 16, zero, 0)
    ones = jnp.ones((16,), jnp.int32)

    def body(i, _):
        o = i * 16
        for s in range(NS):
            lab = lab_v[pl.ds(s * SCHUNK + o, 16)]
            if s:
                lab = lab + jnp.full((16,), s * PAD_C, jnp.int32)
            plsc.addupdate_scatter(h_v, [lab], ones)
        return 0

    lax.fori_loop(0, STEPS, body, 0)
    pltpu.sync_copy(h_v, hist_out.at[pl.ds(w * NS * PAD_C, NS * PAD_C)])


@functools.partial(
    pl.kernel,
    out_type=jax.ShapeDtypeStruct((NW, CELLS), jnp.float32),
    mesh=_mesh,
    compiler_params=_sc_params,
    scratch_types=[
        pltpu.VMEM((CHUNK,), jnp.int32),
        pltpu.VMEM((CHUNK,), jnp.float32),
        pltpu.VMEM((CHUNK,), jnp.float32),
        pltpu.VMEM((NCH * PAD_C,), jnp.int32),
        pltpu.VMEM((PAD_C,), jnp.int32),
        pltpu.VMEM((PAD_C,), jnp.int32),
        pltpu.VMEM((PAD_C,), jnp.int32),
        pltpu.VMEM((PAD_C,), jnp.int32),
        pltpu.VMEM((PAD_C,), jnp.float32),
        pltpu.VMEM((CELLS,), jnp.float32),
        pltpu.SemaphoreType.DMA,
        pltpu.SemaphoreType.DMA,
        pltpu.SemaphoreType.DMA,
    ],
)
def _main_kernel(cf_hbm, ht_hbm, lab_hbm, hist_hbm, accd_out,
                 lab_v, cf_v, ht_v, hist_v, cnt0_v, cnt1_v, cnt2_v, cnt3_v,
                 rcp_v, accd_v, sem1, sem2, sem3):
    w = _wid()
    base = w * CHUNK
    cp1 = pltpu.async_copy(lab_hbm.at[pl.ds(base, CHUNK)], lab_v, sem1)
    cp2 = pltpu.async_copy(cf_hbm.at[pl.ds(base, CHUNK)], cf_v, sem2)
    cp3 = pltpu.async_copy(ht_hbm.at[pl.ds(base, CHUNK)], ht_v, sem3)
    pltpu.sync_copy(hist_hbm, hist_v)

    zi = jnp.zeros((16,), jnp.int32)
    c0 = NS * w
    for j in range(PAD_C // 16):
        def acc_v(v, carry):
            tot, off = carry
            hv = hist_v[pl.ds(v * PAD_C + j * 16, 16)]
            sel = (v < c0).astype(jnp.int32)
            return (tot + hv, off + hv * sel)

        tot, off = lax.fori_loop(0, NCH, acc_v, (zi, zi))
        cnt_refs = (cnt0_v, cnt1_v, cnt2_v, cnt3_v)
        for s in range(NS):
            cnt_refs[s][pl.ds(j * 16, 16)] = off
            if s + 1 < NS:
                off = off + hist_v[pl.ds((c0 + s) * PAD_C + j * 16, 16)]
        m = _div15(tot).astype(jnp.float32)
        rcp_v[pl.ds(j * 16, 16)] = (jnp.float32(1.0) / m) \
            * jnp.float32(1.0 + 2.0 ** -22)

    zf = jnp.zeros((16,), jnp.float32)

    def zero(i, _):
        accd_v[pl.ds(i * 16, 16)] = zf
        return 0

    lax.fori_loop(0, CELLS // 16, zero, 0)
    cp1.wait()
    cp2.wait()
    cp3.wait()

    ones = jnp.ones((16,), jnp.int32)
    tmask = jnp.ones((16,), jnp.bool_)

    def stream(cnt_v, lab, cf, ht):
        cbase = plsc.load_gather(cnt_v, [lab])
        occ, _ = plsc.scan_count(lab, mask=tmask)
        rank = cbase + occ.astype(jnp.int32) - 1
        plsc.addupdate_scatter(cnt_v, [lab], ones)
        r = plsc.load_gather(rcp_v, [lab])
        q = (rank.astype(jnp.float32) * r).astype(jnp.int32)
        q = jnp.clip(q, 0, 15)
        idx = q * PAD_C + lab
        plsc.addupdate_scatter(accd_v, [idx], cf - ht)

    def body(i, _):
        o = i * 16
        for s, cnt_v in enumerate((cnt0_v, cnt1_v, cnt2_v, cnt3_v)):
            so = s * SCHUNK + o
            stream(cnt_v, lab_v[pl.ds(so, 16)], cf_v[pl.ds(so, 16)],
                   ht_v[pl.ds(so, 16)])
        return 0

    lax.fori_loop(0, STEPS, body, 0)
    pltpu.sync_copy(accd_v, accd_out.at[w])


def _final_body(accd_ref, hist_ref, out_ref):
    p = accd_ref[0]
    cnt = hist_ref[0]
    for v in range(1, NW):
        p = p + accd_ref[v]
    for v in range(1, NCH):
        cnt = cnt + hist_ref[v]
    mass = _div15(cnt).astype(jnp.float32)
    coef = jnp.float32(1.0) / (mass * jnp.float32(N_BINS * N_CLASS))
    cls = lax.broadcasted_iota(jnp.int32, (1, PAD_C), 1)
    coef = jnp.where(cls < N_CLASS, coef, jnp.float32(0.0))
    d = jnp.abs(p) * coef
    b = lax.broadcasted_iota(jnp.int32, (16, PAD_C), 0)
    d = jnp.where(b < N_BINS, d, jnp.float32(0.0))
    out_ref[...] = jnp.sum(d).reshape(1, 1)


def kernel(confidences, hits, labels):
    pad = N_PAD - N
    cf = jnp.concatenate([confidences, jnp.zeros((pad,), jnp.float32)])
    ht = jnp.concatenate([hits, jnp.zeros((pad,), jnp.float32)])
    lab = jnp.concatenate([labels, jnp.full((pad,), PAD_C - 1, jnp.int32)])
    hist = _hist_kernel(lab)
    accd = _main_kernel(cf, ht, lab, hist)
    out = pl.pallas_call(
        _final_body,
        out_shape=jax.ShapeDtypeStruct((1, 1), jnp.float32),
    )(accd.reshape(NW, 16, PAD_C), hist.reshape(NCH, 1, PAD_C))
    return out[0, 0]


# speedup vs baseline: 1.8161x; 1.7393x over previous
import functools

import jax
import jax.numpy as jnp
from jax import lax
from jax.experimental import pallas as pl
from jax.experimental.pallas import tpu as pltpu
from jax.experimental.pallas import tpu_sc as plsc

N = 1_000_000
N_CLASS = 100
N_BINS = 15
PAD_C = 128
NW = 32
NS = 4
NCH = NW * NS
SCHUNK = 7_824
CHUNK = NS * SCHUNK
N_PAD = NCH * SCHUNK
STEPS = SCHUNK // 16
CELLS = 16 * PAD_C

_mesh = plsc.VectorSubcoreMesh(core_axis_name="c", subcore_axis_name="s")
_sc_params = pltpu.CompilerParams(needs_layout_passes=False)


def _wid():
    return lax.axis_index("c") * 16 + lax.axis_index("s")


def _div15(cnt):
    m0 = (cnt.astype(jnp.float32) * jnp.float32(1.0 / 15.0)).astype(jnp.int32)
    return m0 + (cnt >= (m0 + 1) * 15).astype(jnp.int32) \
              - (cnt < m0 * 15).astype(jnp.int32)


@functools.partial(
    pl.kernel,
    out_type=jax.ShapeDtypeStruct((NCH * PAD_C,), jnp.int32),
    mesh=_mesh,
    compiler_params=_sc_params,
    scratch_types=[pltpu.VMEM((CHUNK,), jnp.int32),
                   pltpu.VMEM((NS * PAD_C,), jnp.int32)],
)
def _hist_kernel(lab_hbm, hist_out, lab_v, h_v):
    w = _wid()
    pltpu.sync_copy(lab_hbm.at[pl.ds(w * CHUNK, CHUNK)], lab_v)
    zi = jnp.zeros((16,), jnp.int32)

    def zero(i, _):
        h_v[pl.ds(i * 16, 16)] = zi
        return 0

    lax.fori_loop(0, NS * PAD_C // 16, zero, 0)
    ones = jnp.ones((16,), jnp.int32)

    def body(i, _):
        o = i * 16

        @plsc.parallel_loop(0, NS, 1, unroll=NS)
        def _streams(s):
            lab = lab_v[pl.ds(s * SCHUNK + o, 16)] + s * PAD_C
            plsc.addupdate_scatter(h_v, [lab], ones)

        return 0

    lax.fori_loop(0, STEPS, body, 0)
    pltpu.sync_copy(h_v, hist_out.at[pl.ds(w * NS * PAD_C, NS * PAD_C)])


@functools.partial(
    pl.kernel,
    out_type=jax.ShapeDtypeStruct((NW, CELLS), jnp.float32),
    mesh=_mesh,
    compiler_params=_sc_params,
    scratch_types=[
        pltpu.VMEM((CHUNK,), jnp.int32),
        pltpu.VMEM((CHUNK,), jnp.float32),
        pltpu.VMEM((CHUNK,), jnp.float32),
        pltpu.VMEM((NCH * PAD_C,), jnp.int32),
        pltpu.VMEM((NS * PAD_C,), jnp.int32),
        pltpu.VMEM((PAD_C,), jnp.float32),
        pltpu.VMEM((CELLS,), jnp.float32),
        pltpu.SemaphoreType.DMA,
        pltpu.SemaphoreType.DMA,
        pltpu.SemaphoreType.DMA,
    ],
)
def _main_kernel(cf_hbm, ht_hbm, lab_hbm, hist_hbm, accd_out,
                 lab_v, cf_v, ht_v, hist_v, cnt_v,
                 rcp_v, accd_v, sem1, sem2, sem3):
    w = _wid()
    base = w * CHUNK
    cp1 = pltpu.async_copy(lab_hbm.at[pl.ds(base, CHUNK)], lab_v, sem1)
    cp2 = pltpu.async_copy(cf_hbm.at[pl.ds(base, CHUNK)], cf_v, sem2)
    cp3 = pltpu.async_copy(ht_hbm.at[pl.ds(base, CHUNK)], ht_v, sem3)
    pltpu.sync_copy(hist_hbm, hist_v)

    zi = jnp.zeros((16,), jnp.int32)
    c0 = NS * w
    for j in range(PAD_C // 16):
        def acc_v(v, carry):
            tot, off = carry
            hv = hist_v[pl.ds(v * PAD_C + j * 16, 16)]
            sel = (v < c0).astype(jnp.int32)
            return (tot + hv, off + hv * sel)

        tot, off = lax.fori_loop(0, NCH, acc_v, (zi, zi))
        for s in range(NS):
            cnt_v[pl.ds(s * PAD_C + j * 16, 16)] = off
            if s + 1 < NS:
                off = off + hist_v[pl.ds((c0 + s) * PAD_C + j * 16, 16)]
        m = _div15(tot).astype(jnp.float32)
        rcp_v[pl.ds(j * 16, 16)] = (jnp.float32(1.0) / m) \
            * jnp.float32(1.0 + 2.0 ** -22)

    zf = jnp.zeros((16,), jnp.float32)

    def zero(i, _):
        accd_v[pl.ds(i * 16, 16)] = zf
        return 0

    lax.fori_loop(0, CELLS // 16, zero, 0)
    cp1.wait()
    cp2.wait()
    cp3.wait()

    ones = jnp.ones((16,), jnp.int32)
    tmask = jnp.ones((16,), jnp.bool_)

    def body(i, _):
        o = i * 16

        @plsc.parallel_loop(0, NS, 1, unroll=NS)
        def _streams(s):
            so = s * SCHUNK + o
            lab = lab_v[pl.ds(so, 16)]
            cf = cf_v[pl.ds(so, 16)]
            ht = ht_v[pl.ds(so, 16)]
            labc = lab + s * PAD_C
            cbase = plsc.load_gather(cnt_v, [labc])
            occ, _ = plsc.scan_count(lab, mask=tmask)
            rank = cbase + occ.astype(jnp.int32) - 1
            plsc.addupdate_scatter(cnt_v, [labc], ones)
            r = plsc.load_gather(rcp_v, [lab])
            q = (rank.astype(jnp.float32) * r).astype(jnp.int32)
            q = jnp.clip(q, 0, 15)
            idx = q * PAD_C + lab
            plsc.addupdate_scatter(accd_v, [idx], cf - ht)

        return 0

    lax.fori_loop(0, STEPS, body, 0)
    pltpu.sync_copy(accd_v, accd_out.at[w])


def _final_body(accd_ref, hist_ref, out_ref):
    p = accd_ref[0]
    cnt = hist_ref[0]
    for v in range(1, NW):
        p = p + accd_ref[v]
    for v in range(1, NCH):
        cnt = cnt + hist_ref[v]
    mass = _div15(cnt).astype(jnp.float32)
    coef = jnp.float32(1.0) / (mass * jnp.float32(N_BINS * N_CLASS))
    cls = lax.broadcasted_iota(jnp.int32, (1, PAD_C), 1)
    coef = jnp.where(cls < N_CLASS, coef, jnp.float32(0.0))
    d = jnp.abs(p) * coef
    b = lax.broadcasted_iota(jnp.int32, (16, PAD_C), 0)
    d = jnp.where(b < N_BINS, d, jnp.float32(0.0))
    out_ref[...] = jnp.sum(d).reshape(1, 1)


def kernel(confidences, hits, labels):
    pad = N_PAD - N
    cf = jnp.concatenate([confidences, jnp.zeros((pad,), jnp.float32)])
    ht = jnp.concatenate([hits, jnp.zeros((pad,), jnp.float32)])
    lab = jnp.concatenate([labels, jnp.full((pad,), PAD_C - 1, jnp.int32)])
    hist = _hist_kernel(lab)
    accd = _main_kernel(cf, ht, lab, hist)
    out = pl.pallas_call(
        _final_body,
        out_shape=jax.ShapeDtypeStruct((1, 1), jnp.float32),
    )(accd.reshape(NW, 16, PAD_C), hist.reshape(NCH, 1, PAD_C))
    return out[0, 0]


# speedup vs baseline: 1.8670x; 1.0280x over previous
import functools

import jax
import jax.numpy as jnp
from jax import lax
from jax.experimental import pallas as pl
from jax.experimental.pallas import tpu as pltpu
from jax.experimental.pallas import tpu_sc as plsc

N = 1_000_000
N_CLASS = 100
N_BINS = 15
PAD_C = 128
NW = 32
NS = 4
NCH = NW * NS
SCHUNK = 7_808
CHUNK = NS * SCHUNK
N_MAIN = NCH * SCHUNK
TAIL = N - N_MAIN
STEPS = SCHUNK // 16
TAIL_STEPS = TAIL // 16
CELLS = 16 * PAD_C

_mesh = plsc.VectorSubcoreMesh(core_axis_name="c", subcore_axis_name="s")
_sc_params = pltpu.CompilerParams(needs_layout_passes=False)


def _wid():
    return lax.axis_index("c") * 16 + lax.axis_index("s")


def _div15(cnt):
    m0 = (cnt.astype(jnp.float32) * jnp.float32(1.0 / 15.0)).astype(jnp.int32)
    return m0 + (cnt >= (m0 + 1) * 15).astype(jnp.int32) \
              - (cnt < m0 * 15).astype(jnp.int32)


@functools.partial(
    pl.kernel,
    out_type=jax.ShapeDtypeStruct((NCH * PAD_C,), jnp.int32),
    mesh=_mesh,
    compiler_params=_sc_params,
    scratch_types=[pltpu.VMEM((CHUNK,), jnp.int32),
                   pltpu.VMEM((TAIL,), jnp.int32),
                   pltpu.VMEM((NS * PAD_C,), jnp.int32)],
)
def _hist_kernel(lab_hbm, hist_out, lab_v, labt_v, h_v):
    w = _wid()
    pltpu.sync_copy(lab_hbm.at[pl.ds(w * CHUNK, CHUNK)], lab_v)

    @pl.when(w == NW - 1)
    def _copy_tail():
        pltpu.sync_copy(lab_hbm.at[pl.ds(N_MAIN, TAIL)], labt_v)
    zi = jnp.zeros((16,), jnp.int32)

    def zero(i, _):
        h_v[pl.ds(i * 16, 16)] = zi
        return 0

    lax.fori_loop(0, NS * PAD_C // 16, zero, 0)
    ones = jnp.ones((16,), jnp.int32)

    def body(i, _):
        o = i * 16

        @plsc.parallel_loop(0, NS, 1, unroll=NS)
        def _streams(s):
            lab = lab_v[pl.ds(s * SCHUNK + o, 16)] + s * PAD_C
            plsc.addupdate_scatter(h_v, [lab], ones)

        return 0

    lax.fori_loop(0, STEPS, body, 0)

    @pl.when(w == NW - 1)
    def _tail():
        def tbody(i, _):
            lab = labt_v[pl.ds(i * 16, 16)] + (NS - 1) * PAD_C
            plsc.addupdate_scatter(h_v, [lab], ones)
            return 0

        lax.fori_loop(0, TAIL_STEPS, tbody, 0)

    pltpu.sync_copy(h_v, hist_out.at[pl.ds(w * NS * PAD_C, NS * PAD_C)])


@functools.partial(
    pl.kernel,
    out_type=jax.ShapeDtypeStruct((NW, CELLS), jnp.float32),
    mesh=_mesh,
    compiler_params=_sc_params,
    scratch_types=[
        pltpu.VMEM((CHUNK,), jnp.int32),
        pltpu.VMEM((CHUNK,), jnp.float32),
        pltpu.VMEM((CHUNK,), jnp.float32),
        pltpu.VMEM((TAIL,), jnp.int32),
        pltpu.VMEM((TAIL,), jnp.float32),
        pltpu.VMEM((TAIL,), jnp.float32),
        pltpu.VMEM((NCH * PAD_C,), jnp.int32),
        pltpu.VMEM((NS * PAD_C,), jnp.int32),
        pltpu.VMEM((PAD_C,), jnp.float32),
        pltpu.VMEM((CELLS,), jnp.float32),
        pltpu.SemaphoreType.DMA,
        pltpu.SemaphoreType.DMA,
        pltpu.SemaphoreType.DMA,
    ],
)
def _main_kernel(cf_hbm, ht_hbm, lab_hbm, hist_hbm, accd_out,
                 lab_v, cf_v, ht_v, labt_v, cft_v, htt_v, hist_v, cnt_v,
                 rcp_v, accd_v, sem1, sem2, sem3):
    w = _wid()
    base = w * CHUNK
    cp1 = pltpu.async_copy(lab_hbm.at[pl.ds(base, CHUNK)], lab_v, sem1)
    cp2 = pltpu.async_copy(cf_hbm.at[pl.ds(base, CHUNK)], cf_v, sem2)
    cp3 = pltpu.async_copy(ht_hbm.at[pl.ds(base, CHUNK)], ht_v, sem3)

    @pl.when(w == NW - 1)
    def _copy_tail():
        pltpu.sync_copy(lab_hbm.at[pl.ds(N_MAIN, TAIL)], labt_v)
        pltpu.sync_copy(cf_hbm.at[pl.ds(N_MAIN, TAIL)], cft_v)
        pltpu.sync_copy(ht_hbm.at[pl.ds(N_MAIN, TAIL)], htt_v)

    pltpu.sync_copy(hist_hbm, hist_v)

    zi = jnp.zeros((16,), jnp.int32)
    c0 = NS * w
    for j in range(PAD_C // 16):
        def acc_v(v, carry):
            tot, off = carry
            hv = hist_v[pl.ds(v * PAD_C + j * 16, 16)]
            sel = (v < c0).astype(jnp.int32)
            return (tot + hv, off + hv * sel)

        tot, off = lax.fori_loop(0, NCH, acc_v, (zi, zi))
        for s in range(NS):
            cnt_v[pl.ds(s * PAD_C + j * 16, 16)] = off
            if s + 1 < NS:
                off = off + hist_v[pl.ds((c0 + s) * PAD_C + j * 16, 16)]
        m = _div15(tot).astype(jnp.float32)
        rcp_v[pl.ds(j * 16, 16)] = (jnp.float32(1.0) / m) \
            * jnp.float32(1.0 + 2.0 ** -22)

    zf = jnp.zeros((16,), jnp.float32)

    def zero(i, _):
        accd_v[pl.ds(i * 16, 16)] = zf
        return 0

    lax.fori_loop(0, CELLS // 16, zero, 0)
    cp1.wait()
    cp2.wait()
    cp3.wait()

    ones = jnp.ones((16,), jnp.int32)
    tmask = jnp.ones((16,), jnp.bool_)

    def sample16(lab, cf, ht, creg):
        labc = lab + creg
        cbase = plsc.load_gather(cnt_v, [labc])
        occ, _ = plsc.scan_count(lab, mask=tmask)
        rank = cbase + occ.astype(jnp.int32) - 1
        plsc.addupdate_scatter(cnt_v, [labc], ones)
        r = plsc.load_gather(rcp_v, [lab])
        q = (rank.astype(jnp.float32) * r).astype(jnp.int32)
        q = jnp.clip(q, 0, 15)
        idx = q * PAD_C + lab
        plsc.addupdate_scatter(accd_v, [idx], cf - ht)

    def body(i, _):
        o = i * 16

        @plsc.parallel_loop(0, NS, 1, unroll=NS)
        def _streams(s):
            so = s * SCHUNK + o
            sample16(lab_v[pl.ds(so, 16)], cf_v[pl.ds(so, 16)],
                     ht_v[pl.ds(so, 16)], s * PAD_C)

        return 0

    lax.fori_loop(0, STEPS, body, 0)

    @pl.when(w == NW - 1)
    def _tail():
        def tbody(i, _):
            o = i * 16
            sample16(labt_v[pl.ds(o, 16)], cft_v[pl.ds(o, 16)],
                     htt_v[pl.ds(o, 16)], (NS - 1) * PAD_C)
            return 0

        lax.fori_loop(0, TAIL_STEPS, tbody, 0)

    pltpu.sync_copy(accd_v, accd_out.at[w])


def _final_body(accd_ref, hist_ref, out_ref):
    p = accd_ref[0]
    cnt = hist_ref[0]
    for v in range(1, NW):
        p = p + accd_ref[v]
    for v in range(1, NCH):
        cnt = cnt + hist_ref[v]
    mass = _div15(cnt).astype(jnp.float32)
    coef = jnp.float32(1.0) / (mass * jnp.float32(N_BINS * N_CLASS))
    cls = lax.broadcasted_iota(jnp.int32, (1, PAD_C), 1)
    coef = jnp.where(cls < N_CLASS, coef, jnp.float32(0.0))
    d = jnp.abs(p) * coef
    b = lax.broadcasted_iota(jnp.int32, (16, PAD_C), 0)
    d = jnp.where(b < N_BINS, d, jnp.float32(0.0))
    out_ref[...] = jnp.sum(d).reshape(1, 1)


def kernel(confidences, hits, labels):
    hist = _hist_kernel(labels)
    accd = _main_kernel(confidences, hits, labels, hist)
    out = pl.pallas_call(
        _final_body,
        out_shape=jax.ShapeDtypeStruct((1, 1), jnp.float32),
    )(accd.reshape(NW, 16, PAD_C), hist.reshape(NCH, 1, PAD_C))
    return out[0, 0]


# speedup vs baseline: 1.9014x; 1.0185x over previous
import functools

import jax
import jax.numpy as jnp
from jax import lax
from jax.experimental import pallas as pl
from jax.experimental.pallas import tpu as pltpu
from jax.experimental.pallas import tpu_sc as plsc

N = 1_000_000
N_CLASS = 100
N_BINS = 15
PAD_C = 128
NW = 32
NS = 4
NCH = NW * NS
HC = 112
SCHUNK = 7_808
CHUNK = NS * SCHUNK
N_MAIN = NCH * SCHUNK
TAIL = N - N_MAIN
STEPS = SCHUNK // 16
TAIL_STEPS = TAIL // 16
CELLS = 16 * PAD_C

_mesh = plsc.VectorSubcoreMesh(core_axis_name="c", subcore_axis_name="s")
_sc_params = pltpu.CompilerParams(needs_layout_passes=False)


def _wid():
    return lax.axis_index("c") * 16 + lax.axis_index("s")


def _div15(cnt):
    m0 = (cnt.astype(jnp.float32) * jnp.float32(1.0 / 15.0)).astype(jnp.int32)
    return m0 + (cnt >= (m0 + 1) * 15).astype(jnp.int32) \
              - (cnt < m0 * 15).astype(jnp.int32)


@functools.partial(
    pl.kernel,
    out_type=jax.ShapeDtypeStruct((NCH * HC,), jnp.int32),
    mesh=_mesh,
    compiler_params=_sc_params,
    scratch_types=[pltpu.VMEM((CHUNK,), jnp.int32),
                   pltpu.VMEM((TAIL,), jnp.int32),
                   pltpu.VMEM((NS * HC,), jnp.int32)],
)
def _hist_kernel(lab_hbm, hist_out, lab_v, labt_v, h_v):
    w = _wid()
    pltpu.sync_copy(lab_hbm.at[pl.ds(w * CHUNK, CHUNK)], lab_v)

    @pl.when(w == NW - 1)
    def _copy_tail():
        pltpu.sync_copy(lab_hbm.at[pl.ds(N_MAIN, TAIL)], labt_v)
    zi = jnp.zeros((16,), jnp.int32)

    def zero(i, _):
        h_v[pl.ds(i * 16, 16)] = zi
        return 0

    lax.fori_loop(0, NS * HC // 16, zero, 0)
    ones = jnp.ones((16,), jnp.int32)

    def body(i, _):
        o = i * 16

        @plsc.parallel_loop(0, NS, 1, unroll=NS)
        def _streams(s):
            lab = lab_v[pl.ds(s * SCHUNK + o, 16)] + s * HC
            plsc.addupdate_scatter(h_v, [lab], ones)

        return 0

    lax.fori_loop(0, STEPS, body, 0)

    @pl.when(w == NW - 1)
    def _tail():
        def tbody(i, _):
            lab = labt_v[pl.ds(i * 16, 16)] + (NS - 1) * HC
            plsc.addupdate_scatter(h_v, [lab], ones)
            return 0

        lax.fori_loop(0, TAIL_STEPS, tbody, 0)

    pltpu.sync_copy(h_v, hist_out.at[pl.ds(w * NS * HC, NS * HC)])


@functools.partial(
    pl.kernel,
    out_type=jax.ShapeDtypeStruct((NW, CELLS), jnp.float32),
    mesh=_mesh,
    compiler_params=_sc_params,
    scratch_types=[
        pltpu.VMEM((CHUNK,), jnp.int32),
        pltpu.VMEM((CHUNK,), jnp.float32),
        pltpu.VMEM((CHUNK,), jnp.float32),
        pltpu.VMEM((TAIL,), jnp.int32),
        pltpu.VMEM((TAIL,), jnp.float32),
        pltpu.VMEM((TAIL,), jnp.float32),
        pltpu.VMEM((NCH * HC,), jnp.int32),
        pltpu.VMEM((NS * HC,), jnp.int32),
        pltpu.VMEM((HC,), jnp.float32),
        pltpu.VMEM((CELLS,), jnp.float32),
        pltpu.SemaphoreType.DMA,
        pltpu.SemaphoreType.DMA,
        pltpu.SemaphoreType.DMA,
    ],
)
def _main_kernel(cf_hbm, ht_hbm, lab_hbm, hist_hbm, accd_out,
                 lab_v, cf_v, ht_v, labt_v, cft_v, htt_v, hist_v, cnt_v,
                 rcp_v, accd_v, sem1, sem2, sem3):
    w = _wid()
    base = w * CHUNK
    cp1 = pltpu.async_copy(lab_hbm.at[pl.ds(base, CHUNK)], lab_v, sem1)
    cp2 = pltpu.async_copy(cf_hbm.at[pl.ds(base, CHUNK)], cf_v, sem2)
    cp3 = pltpu.async_copy(ht_hbm.at[pl.ds(base, CHUNK)], ht_v, sem3)

    @pl.when(w == NW - 1)
    def _copy_tail():
        pltpu.sync_copy(lab_hbm.at[pl.ds(N_MAIN, TAIL)], labt_v)
        pltpu.sync_copy(cf_hbm.at[pl.ds(N_MAIN, TAIL)], cft_v)
        pltpu.sync_copy(ht_hbm.at[pl.ds(N_MAIN, TAIL)], htt_v)

    pltpu.sync_copy(hist_hbm, hist_v)

    zi = jnp.zeros((16,), jnp.int32)
    c0 = NS * w

    @plsc.parallel_loop(0, HC // 16, 1, unroll=HC // 16)
    def _groups(j):
        def acc_v(v, carry):
            tot, off = carry
            hv = hist_v[pl.ds(v * HC + j * 16, 16)]
            sel = (v < c0).astype(jnp.int32)
            return (tot + hv, off + hv * sel)

        tot, off = lax.fori_loop(0, NCH, acc_v, (zi, zi))
        for s in range(NS):
            cnt_v[pl.ds(s * HC + j * 16, 16)] = off
            if s + 1 < NS:
                off = off + hist_v[pl.ds((c0 + s) * HC + j * 16, 16)]
        m = _div15(tot).astype(jnp.float32)
        rcp_v[pl.ds(j * 16, 16)] = (jnp.float32(1.0) / m) \
            * jnp.float32(1.0 + 2.0 ** -22)

    zf = jnp.zeros((16,), jnp.float32)

    def zero(i, _):
        accd_v[pl.ds(i * 16, 16)] = zf
        return 0

    lax.fori_loop(0, CELLS // 16, zero, 0)
    cp1.wait()
    cp2.wait()
    cp3.wait()

    ones = jnp.ones((16,), jnp.int32)
    tmask = jnp.ones((16,), jnp.bool_)

    def sample16(lab, cf, ht, creg):
        labc = lab + creg
        cbase = plsc.load_gather(cnt_v, [labc])
        occ, _ = plsc.scan_count(lab, mask=tmask)
        rank = cbase + occ.astype(jnp.int32) - 1
        plsc.addupdate_scatter(cnt_v, [labc], ones)
        r = plsc.load_gather(rcp_v, [lab])
        q = (rank.astype(jnp.float32) * r).astype(jnp.int32)
        q = jnp.clip(q, 0, 15)
        idx = q * PAD_C + lab
        plsc.addupdate_scatter(accd_v, [idx], cf - ht)

    def body(i, _):
        o = i * 16

        @plsc.parallel_loop(0, NS, 1, unroll=NS)
        def _streams(s):
            so = s * SCHUNK + o
            sample16(lab_v[pl.ds(so, 16)], cf_v[pl.ds(so, 16)],
                     ht_v[pl.ds(so, 16)], s * HC)

        return 0

    lax.fori_loop(0, STEPS, body, 0)

    @pl.when(w == NW - 1)
    def _tail():
        def tbody(i, _):
            o = i * 16
            sample16(labt_v[pl.ds(o, 16)], cft_v[pl.ds(o, 16)],
                     htt_v[pl.ds(o, 16)], (NS - 1) * HC)
            return 0

        lax.fori_loop(0, TAIL_STEPS, tbody, 0)

    pltpu.sync_copy(accd_v, accd_out.at[w])


def _final_body(accd_ref, hist_ref, out_ref):
    p = accd_ref[0]
    cnt = hist_ref[0]
    for v in range(1, NW):
        p = p + accd_ref[v]
    for v in range(1, NCH):
        cnt = cnt + hist_ref[v]
    mass = _div15(cnt).astype(jnp.float32)
    coef = jnp.float32(1.0) / (mass * jnp.float32(N_BINS * N_CLASS))
    cls = lax.broadcasted_iota(jnp.int32, (1, HC), 1)
    coef = jnp.where(cls < N_CLASS, coef, jnp.float32(0.0))
    coef = jnp.concatenate(
        [coef, jnp.zeros((1, PAD_C - HC), jnp.float32)], axis=1)
    d = jnp.abs(p) * coef
    b = lax.broadcasted_iota(jnp.int32, (16, PAD_C), 0)
    d = jnp.where(b < N_BINS, d, jnp.float32(0.0))
    out_ref[...] = jnp.sum(d).reshape(1, 1)


def kernel(confidences, hits, labels):
    hist = _hist_kernel(labels)
    accd = _main_kernel(confidences, hits, labels, hist)
    out = pl.pallas_call(
        _final_body,
        out_shape=jax.ShapeDtypeStruct((1, 1), jnp.float32),
    )(accd.reshape(NW, 16, PAD_C), hist.reshape(NCH, 1, HC))
    return out[0, 0]
